# baseline (speedup 1.0000x reference)
import functools

import jax
import jax.numpy as jnp
from jax import lax
from jax.experimental import pallas as pl
from jax.experimental.pallas import tpu as pltpu
from jax.experimental.pallas import tpu_sc as plsc

_B, _H, _S, _D = 4, 16, 4096, 128
_N = _B * _H * _S
_NW = 32
_S_SC = 2048
_S_TC = _S - _S_SC
_WR = _S_SC // 4
_RPW = (8 * _WR) // 128


def _sc_select(x1, idxv):
    mesh = plsc.VectorSubcoreMesh(core_axis_name="c", subcore_axis_name="s")

    @functools.partial(
        pl.kernel,
        mesh=mesh,
        out_type=jax.ShapeDtypeStruct((_B, _H, _S_SC), jnp.float32),
        scratch_types=[
            pltpu.VMEM((16,), jnp.int32),
            pltpu.VMEM((_RPW, 128), jnp.int32),
            pltpu.VMEM((8, _WR), jnp.float32),
            pltpu.SemaphoreType.DMA,
        ],
        compiler_params=pltpu.CompilerParams(
            use_tc_tiling_on_sc=True,
            disable_bounds_checks=True,
            disable_semaphore_checks=True,
        ),
    )
    def k(x_hbm, idx_hbm, out_hbm, idx_v, gidx, buf, sem):
        wid = lax.axis_index("s") * 2 + lax.axis_index("c")
        pltpu.sync_copy(idx_hbm, idx_v)
        vidx = idx_v[...]
        lane = lax.iota(jnp.int32, 16) * _D

        b = wid // 8
        w8 = wid % 8
        h0 = (w8 // 4) * 8
        s0 = (w8 % 4) * _WR
        nsb = _WR // 128

        def fill_and_fire(j, carry):
            r, c = j // nsb, j % nsb
            base = ((b * _H + h0 + r) * _S + s0 + c * 128) * _D + vidx
            for kk in range(8):
                gidx[j, pl.ds(kk * 16, 16)] = base + (kk * 16 * _D) + lane
            pltpu.async_copy(
                x_hbm.at[gidx.at[j]],
                buf.at[r, pl.ds(c * 128, 128)],
                sem,
            )
            return carry

        lax.fori_loop(0, _RPW, fill_and_fire, 0)
        out_blk = out_hbm.at[b, pl.ds(h0, 8), pl.ds(s0, _WR)]
        pltpu.make_async_copy(out_blk, buf, sem).wait()
        pltpu.sync_copy(buf, out_blk)

    return k(x1, idxv)


def _tc_body(idx_ref, x_ref, o_ref):
    idx = idx_ref[0]
    xb = x_ref[0]
    lanes = lax.broadcasted_iota(jnp.int32, xb.shape, 2)
    o_ref[0] = jnp.sum(jnp.where(lanes == idx, xb, 0.0), axis=2)


def _tc_select(x, idx_arr):
    sb0 = _S_SC // 512
    return pl.pallas_call(
        _tc_body,
        grid_spec=pltpu.PrefetchScalarGridSpec(
            num_scalar_prefetch=1,
            grid=(_B, _H // 8, _S_TC // 512),
            in_specs=[
                pl.BlockSpec(
                    (1, 8, 512, _D), lambda b, hb, s, idx_ref: (b, hb, s + sb0, 0)
                )
            ],
            out_specs=pl.BlockSpec(
                (1, 8, 512), lambda b, hb, s, idx_ref: (b, hb, s)
            ),
        ),
        out_shape=jax.ShapeDtypeStruct((_B, _H, _S_TC), jnp.float32),
        compiler_params=pltpu.CompilerParams(
            dimension_semantics=("parallel", "parallel", "arbitrary"),
        ),
    )(idx_arr, x)


@jax.jit
def _select(x, idx):
    x1 = x.reshape(_N * _D)
    idxv = jnp.full((16,), idx, jnp.int32)
    sc_out = _sc_select(x1, idxv)
    tc_out = _tc_select(x, idx.reshape(1))
    return jnp.concatenate([sc_out, tc_out], axis=2)


def kernel(x, dim, index):
    idx = (jnp.asarray(index) + jnp.asarray(dim) - 3).astype(jnp.int32)
    return _select(x, idx)


# speedup vs baseline: 1.9138x; 1.9138x over previous
import functools

import jax
import jax.numpy as jnp
from jax import lax
from jax.experimental import pallas as pl
from jax.experimental.pallas import tpu as pltpu
from jax.experimental.pallas import tpu_sc as plsc

_B, _H, _S, _D = 4, 16, 4096, 128
_N = _B * _H * _S
_NW = 32
_PER = _N // _NW
_ROWS = _PER // 128


@jax.jit
def _sc_select(x1, idxv):
    mesh = plsc.VectorSubcoreMesh(core_axis_name="c", subcore_axis_name="s")

    @functools.partial(
        pl.kernel,
        mesh=mesh,
        out_type=jax.ShapeDtypeStruct((_B, _H, _S), jnp.float32),
        scratch_types=[
            pltpu.VMEM((16,), jnp.int32),
            pltpu.VMEM((_ROWS, 128), jnp.int32),
            pltpu.VMEM((8, 1024), jnp.float32),
            pltpu.SemaphoreType.DMA,
        ],
        compiler_params=pltpu.CompilerParams(
            use_tc_tiling_on_sc=True,
            disable_bounds_checks=True,
            disable_semaphore_checks=True,
        ),
    )
    def k(x_hbm, idx_hbm, out_hbm, idx_v, gidx, buf, sem):
        wid = lax.axis_index("s") * 2 + lax.axis_index("c")
        pltpu.sync_copy(idx_hbm, idx_v)
        vidx = idx_v[...]
        lane = lax.iota(jnp.int32, 16) * _D

        b = wid // 8
        w8 = wid % 8
        h0 = (w8 // 4) * 8
        s0 = (w8 % 4) * 1024

        def fill_and_fire(j2, carry):
            for u in range(2):
                j = j2 * 2 + u
                r, c = j // 8, j % 8
                base = ((b * _H + h0 + r) * _S + s0 + c * 128) * _D + vidx
                for kk in range(8):
                    gidx[j, pl.ds(kk * 16, 16)] = base + (kk * 16 * _D) + lane
                pltpu.async_copy(
                    x_hbm.at[gidx.at[j]],
                    buf.at[r, pl.ds(c * 128, 128)],
                    sem,
                )
            return carry

        lax.fori_loop(0, _ROWS // 2, fill_and_fire, 0)
        out_blk = out_hbm.at[b, pl.ds(h0, 8), pl.ds(s0, 1024)]
        pltpu.make_async_copy(out_blk, buf, sem).wait()
        pltpu.sync_copy(buf, out_blk)

    return k(x1, idxv)


def kernel(x, dim, index):
    idx = (jnp.asarray(index) + jnp.asarray(dim) - 3).astype(jnp.int32)
    x1 = x.reshape(_N * _D)
    return _sc_select(x1, jnp.full((16,), idx, jnp.int32))


# speedup vs baseline: 1.9313x; 1.0092x over previous
import functools

import jax
import jax.numpy as jnp
from jax import lax
from jax.experimental import pallas as pl
from jax.experimental.pallas import tpu as pltpu
from jax.experimental.pallas import tpu_sc as plsc

_B, _H, _S, _D = 4, 16, 4096, 128
_N = _B * _H * _S
_NW = 32
_PER = _N // _NW
_ROWS = _PER // 128


@jax.jit
def _sc_select(x1, idxv):
    mesh = plsc.VectorSubcoreMesh(core_axis_name="c", subcore_axis_name="s")

    @functools.partial(
        pl.kernel,
        mesh=mesh,
        out_type=jax.ShapeDtypeStruct((_B, _H, _S), jnp.float32),
        scratch_types=[
            pltpu.VMEM((16,), jnp.int32),
            pltpu.VMEM((_ROWS, 128), jnp.int32),
            pltpu.VMEM((8, 1024), jnp.float32),
            pltpu.SemaphoreType.DMA,
        ],
        compiler_params=pltpu.CompilerParams(
            use_tc_tiling_on_sc=True,
            disable_bounds_checks=True,
            disable_semaphore_checks=True,
        ),
    )
    def k(x_hbm, idx_hbm, out_hbm, idx_v, gidx, buf, sem):
        wid = lax.axis_index("s") * 2 + lax.axis_index("c")
        pltpu.sync_copy(idx_hbm, idx_v)
        vidx = idx_v[...]
        lane = lax.iota(jnp.int32, 16) * _D

        b = wid // 8
        w8 = wid % 8
        h0 = (w8 // 4) * 8
        s0 = (w8 % 4) * 1024

        def fill_and_fire(j, carry):
            r, c = j // 8, j % 8
            base = ((b * _H + h0 + r) * _S + s0 + c * 128) * _D + vidx
            for kk in range(8):
                gidx[j, pl.ds(kk * 16, 16)] = base + (kk * 16 * _D) + lane
            pltpu.async_copy(
                x_hbm.at[gidx.at[j]],
                buf.at[r, pl.ds(c * 128, 128)],
                sem,
            )
            return carry

        lax.fori_loop(0, _ROWS, fill_and_fire, 0)
        out_blk = out_hbm.at[b, pl.ds(h0, 8), pl.ds(s0, 1024)]
        pltpu.make_async_copy(out_blk, buf, sem).wait()
        pltpu.sync_copy(buf, out_blk)

    return k(x1, idxv)


def kernel(x, dim, index):
    idx = (jnp.asarray(index) + jnp.asarray(dim) - 3).astype(jnp.int32)
    x1 = x.reshape(_N * _D)
    return _sc_select(x1, jnp.full((16,), idx, jnp.int32))
